# baseline (speedup 1.0000x reference)
import functools

import jax
import jax.numpy as jnp
from jax import lax
from jax.experimental import pallas as pl
from jax.experimental.pallas import tpu as pltpu
from jax.experimental.pallas import tpu_sc as plsc

N = 100000
E = 1600000
IN_FEATS = 128
H = 32
HH = 16
PROP_STEP = 8

R = 100480
DUMP = N
NT = 16
NR = R // NT

CH = 4
EP = 1605632
EROWS = EP // 128
RPT = EROWS // NT
CHUNKS = RPT // CH

BLK = 2048
GRID = (N + BLK - 1) // BLK

RP = R // 8
PBLK = 1024
PGRID = (RP + PBLK - 1) // PBLK

_MESH = plsc.VectorSubcoreMesh(core_axis_name="c", subcore_axis_name="s")
_SC_PARAMS = pltpu.CompilerParams(use_tc_tiling_on_sc=False)


def _zero_acc(acc, zn, s):
    pltpu.sync_copy(zn, acc.at[pl.ds(s * NR, NR)])


def _sc_degrees_body(srcr, dstr, zn, degd, degs, acc, idx, ones, sem):
    c = lax.axis_index("c")
    s = lax.axis_index("s")
    _zero_acc(acc, zn, s)

    def of(i, _):
        ones[i, :] = jnp.ones((HH,), jnp.float32)
        return _

    lax.fori_loop(0, 128, of, None)
    plsc.subcore_barrier()
    base = s * NR
    for c_static, (idxsrc, out) in enumerate(((dstr, degd), (srcr, degs))):
        @pl.when(c == c_static)
        def _():
            t0 = s * RPT
            pltpu.sync_copy(idxsrc.at[pl.ds(t0, CH)], idx.at[0])

            def chunk(k, _):
                p = lax.rem(k, 2)
                q = 1 - p

                @pl.when(k >= 1)
                def _():
                    for j in range(CH):
                        pltpu.make_async_copy(
                            ones, acc.at[idx.at[q, j]], sem).wait()

                @pl.when(k < CHUNKS - 1)
                def _():
                    pltpu.sync_copy(
                        idxsrc.at[pl.ds(t0 + (k + 1) * CH, CH)], idx.at[q])

                for j in range(CH):
                    pltpu.async_copy(ones, acc.at[idx.at[p, j]], sem,
                                     add=True)
                return _

            lax.fori_loop(0, CHUNKS, chunk, None)
            pf = (CHUNKS - 1) % 2
            for j in range(CH):
                pltpu.make_async_copy(ones, acc.at[idx.at[pf, j]], sem).wait()
            plsc.subcore_barrier()
            pltpu.sync_copy(acc.at[pl.ds(base, NR)], out.at[pl.ds(base, NR)])


_sc_degrees = pl.kernel(
    _sc_degrees_body,
    out_type=[
        jax.ShapeDtypeStruct((R, HH), jnp.float32),
        jax.ShapeDtypeStruct((R, HH), jnp.float32),
    ],
    mesh=_MESH,
    scratch_types=[
        pltpu.VMEM_SHARED((R, HH), jnp.float32),
        pltpu.VMEM((2, CH, 128), jnp.int32),
        pltpu.VMEM((128, HH), jnp.float32),
        pltpu.SemaphoreType.DMA,
    ],
    compiler_params=_SC_PARAMS,
)


CHW = CH // 2
RPTW = RPT // 2


def _sc_agg_body(g0, g1, srcw, dstr, zn, a0, a1, acc, sidx, didx, rows,
                 semg, sems):
    c = lax.axis_index("c")
    s = lax.axis_index("s")
    _zero_acc(acc, zn, s)
    plsc.subcore_barrier()
    base = s * NR

    def gathers(g, b, jj):
        for j in range(CHW):
            pltpu.async_copy(g.at[sidx.at[b, j]], rows.at[b, j], semg)
        del jj

    for c_static, (g, a) in enumerate(((g0, a0), (g1, a1))):
        @pl.when(c == c_static)
        def _():
            t0 = s * RPTW
            t0d = s * RPT
            pltpu.sync_copy(srcw.at[pl.ds(t0, CHW)], sidx.at[0])
            pltpu.sync_copy(dstr.at[pl.ds(t0d, CH)], didx.at[0])
            gathers(g, 0, 0)

            def chunk(k, _):
                p = lax.rem(k, 3)
                q = lax.rem(k + 1, 3)

                @pl.when(k >= 2)
                def _():
                    for j in range(CHW):
                        for h in range(2):
                            pltpu.make_async_copy(
                                rows.at[q, j, pl.ds(h * 128, 128)],
                                acc.at[didx.at[q, 2 * j + h]], sems).wait()

                @pl.when(k < CHUNKS - 1)
                def _():
                    pltpu.sync_copy(
                        srcw.at[pl.ds(t0 + (k + 1) * CHW, CHW)], sidx.at[q])
                    pltpu.sync_copy(
                        dstr.at[pl.ds(t0d + (k + 1) * CH, CH)], didx.at[q])

                for j in range(CHW):
                    pltpu.make_async_copy(
                        g.at[sidx.at[p, j]], rows.at[p, j], semg).wait()

                @pl.when(k < CHUNKS - 1)
                def _():
                    gathers(g, q, 0)

                for j in range(CHW):
                    for h in range(2):
                        pltpu.async_copy(
                            rows.at[p, j, pl.ds(h * 128, 128)],
                            acc.at[didx.at[p, 2 * j + h]], sems, add=True)
                return _

            lax.fori_loop(0, CHUNKS, chunk, None)
            for k in (CHUNKS - 2, CHUNKS - 1):
                pf = k % 3
                for j in range(CHW):
                    for h in range(2):
                        pltpu.make_async_copy(
                            rows.at[pf, j, pl.ds(h * 128, 128)],
                            acc.at[didx.at[pf, 2 * j + h]], sems).wait()
            plsc.subcore_barrier()
            pltpu.sync_copy(acc.at[pl.ds(base, NR)], a.at[pl.ds(base, NR)])


_sc_agg = pl.kernel(
    _sc_agg_body,
    out_type=[
        jax.ShapeDtypeStruct((R, HH), jnp.float32),
        jax.ShapeDtypeStruct((R, HH), jnp.float32),
    ],
    mesh=_MESH,
    scratch_types=[
        pltpu.VMEM_SHARED((R, HH), jnp.float32),
        pltpu.VMEM((3, CHW, 256), jnp.int32),
        pltpu.VMEM((3, CH, 128), jnp.int32),
        pltpu.VMEM((3, CHW, 256, HH), jnp.float32),
        pltpu.SemaphoreType.DMA,
        pltpu.SemaphoreType.DMA,
    ],
    compiler_params=_SC_PARAMS,
)


def _nrm(deg_ref):
    return lax.rsqrt(jnp.maximum(deg_ref[:, :1], 1.0))


def _tc_prep_body(x_ref, w1_ref, g0_ref, g1_ref):
    h = jnp.dot(x_ref[...], w1_ref[...], preferred_element_type=jnp.float32)
    g0_ref[...] = h[:, :HH]
    g1_ref[...] = h[:, HH:]


def _tc_pack_body(degd_ref, degs_ref, cp_ref, np_ref):
    nsrc = lax.rsqrt(jnp.maximum(degs_ref[...], 1.0))
    ndst = lax.rsqrt(jnp.maximum(degd_ref[...], 1.0))
    cp_ref[...] = nsrc * ndst
    np_ref[...] = nsrc


def _tc_scale_body(a0_ref, a1_ref, cp_ref, g0_ref, g1_ref):
    c = cp_ref[...]
    g0_ref[...] = a0_ref[...] * c
    g1_ref[...] = a1_ref[...] * c


def _tc_last_body(a0_ref, a1_ref, degd_ref, w_ref, b_ref, out_ref):
    ndst = _nrm(degd_ref)
    a = jnp.concatenate([a0_ref[...], a1_ref[...]], axis=1)
    w = w_ref[...]
    w7 = w
    for _ in range(PROP_STEP - 2):
        w7 = jnp.dot(w7, w, preferred_element_type=jnp.float32)
    out_ref[...] = (
        jnp.dot(a, w7, preferred_element_type=jnp.float32) * ndst
        + b_ref[...]
    )


def _half_spec():
    return pl.BlockSpec((BLK, HH), lambda i: (i, 0))


def _full_spec(shape):
    return pl.BlockSpec(shape, lambda i: (0, 0))


_tc_prep = pl.pallas_call(
    _tc_prep_body,
    grid=(GRID,),
    in_specs=[
        pl.BlockSpec((BLK, IN_FEATS), lambda i: (i, 0)),
        _full_spec((IN_FEATS, H)),
    ],
    out_specs=[_half_spec(), _half_spec()],
    out_shape=[
        jax.ShapeDtypeStruct((R, HH), jnp.float32),
        jax.ShapeDtypeStruct((R, HH), jnp.float32),
    ],
)

def _packed_spec():
    return pl.BlockSpec((PBLK, 128), lambda i: (i, 0))


_tc_pack = pl.pallas_call(
    _tc_pack_body,
    grid=(PGRID,),
    in_specs=[_packed_spec(), _packed_spec()],
    out_specs=[_packed_spec(), _packed_spec()],
    out_shape=[
        jax.ShapeDtypeStruct((RP, 128), jnp.float32),
        jax.ShapeDtypeStruct((RP, 128), jnp.float32),
    ],
)

_tc_scale = pl.pallas_call(
    _tc_scale_body,
    grid=(PGRID,),
    in_specs=[_packed_spec(), _packed_spec(), _packed_spec()],
    out_specs=[_packed_spec(), _packed_spec()],
    out_shape=[
        jax.ShapeDtypeStruct((RP, 128), jnp.float32),
        jax.ShapeDtypeStruct((RP, 128), jnp.float32),
    ],
)

_tc_last = pl.pallas_call(
    _tc_last_body,
    grid=(GRID,),
    in_specs=[_half_spec(), _half_spec(), _half_spec(),
              _full_spec((H, H)), _full_spec((1, H))],
    out_specs=pl.BlockSpec((BLK, H), lambda i: (i, 0)),
    out_shape=jax.ShapeDtypeStruct((N, H), jnp.float32),
)


def kernel(in_feat, edge_index, W1, b1, W2, b2):
    pad = jnp.full((EP - E,), DUMP, jnp.int32)
    srcf = jnp.concatenate([edge_index[0], pad])
    srcr = srcf.reshape(EROWS, 128)
    srcw = srcf.reshape(EROWS // 2, 256)
    dstr = jnp.concatenate([edge_index[1], pad]).reshape(EROWS, 128)
    zn = jnp.zeros((NR, HH), jnp.float32)
    b1r = b1.reshape(1, H)
    b2r = b2.reshape(1, H)

    degd, degs = _sc_degrees(srcr, dstr, zn)
    cp, nsp = _tc_pack(degd.reshape(RP, 128), degs.reshape(RP, 128))
    p0, p1 = _tc_prep(in_feat, W1)
    g0p, g1p = _tc_scale(p0.reshape(RP, 128), p1.reshape(RP, 128), nsp)
    g0 = g0p.reshape(R, HH)
    g1 = g1p.reshape(R, HH)
    for _ in range(PROP_STEP - 1):
        a0, a1 = _sc_agg(g0, g1, srcw, dstr, zn)
        g0p, g1p = _tc_scale(a0.reshape(RP, 128), a1.reshape(RP, 128), cp)
        g0 = g0p.reshape(R, HH)
        g1 = g1p.reshape(R, HH)
    a0, a1 = _sc_agg(g0, g1, srcw, dstr, zn)
    return _tc_last(a0, a1, degd, W2, b2r)
